# baseline (speedup 1.0000x reference)
import functools
import jax
import jax.numpy as jnp
from jax import lax
from jax.experimental import pallas as pl
from jax.experimental.pallas import tpu as pltpu
from jax.experimental.pallas import tpu_sc as plsc

_NC = 2
_NS = 16
_NW = _NC * _NS

_H = 32
_C = 128
_K = 15
_CPG = 16
_GROUPS = 8
_SIGMA = 2.0

_R = 8
_E = _R * _H

_CHUNKS0 = 62
_CHUNKS1 = 18


def _mods_tc(sf_pad, W1, b1r, W2p):
    NP = sf_pad.shape[0]
    BR = 256

    def body(x_ref, w1_ref, b1_ref, w2_ref, o_ref):
        h = jnp.dot(x_ref[...], w1_ref[...], preferred_element_type=jnp.float32)
        h = h + b1_ref[...]
        h = jnp.where(h >= 0.0, h, 0.1 * h)
        m = jnp.dot(h, w2_ref[...], preferred_element_type=jnp.float32)
        o_ref[...] = jax.nn.sigmoid(m)

    return pl.pallas_call(
        body,
        grid=(NP // BR,),
        in_specs=[
            pl.BlockSpec((BR, _C), lambda i: (i, 0)),
            pl.BlockSpec((_C, _C), lambda i: (0, 0)),
            pl.BlockSpec((1, _C), lambda i: (0, 0)),
            pl.BlockSpec((_C, 256), lambda i: (0, 0)),
        ],
        out_specs=pl.BlockSpec((BR, 256), lambda i: (i, 0)),
        out_shape=jax.ShapeDtypeStruct((NP, 256), jnp.float32),
    )(sf_pad, W1, b1r, W2p)


def _make_sc(N, NP, chunks0, chunks1):
    assert 16 * (chunks0 + chunks1) * _R == NP
    assert chunks0 % 2 == 0 and chunks1 % 2 == 0
    mesh = plsc.VectorSubcoreMesh(core_axis_name="c", subcore_axis_name="s",
                                  num_cores=_NC, num_subcores=_NS)

    slot_types = [
        pltpu.VMEM((2, 128), jnp.int32),
        pltpu.VMEM((_E, _C), jnp.float32),
        pltpu.VMEM((_R, 256), jnp.float32),
        pltpu.VMEM((_R, 16), jnp.float32),
        pltpu.VMEM((_E + 16,), jnp.int32),
        pltpu.VMEM((_R, _C), jnp.float32),
        pltpu.SemaphoreType.DMA,
        pltpu.SemaphoreType.DMA,
        pltpu.SemaphoreType.DMA,
    ]

    @functools.partial(
        pl.kernel,
        out_type=jax.ShapeDtypeStruct((NP, _C), jnp.float32),
        mesh=mesh,
        scratch_types=[
            pltpu.VMEM((N,), jnp.float32),
            pltpu.VMEM((N,), jnp.float32),
            pltpu.VMEM((N,), jnp.float32),
            pltpu.VMEM((_K, _C), jnp.float32),
            pltpu.VMEM((48,), jnp.float32),
        ] + slot_types + slot_types,
        compiler_params=pltpu.CompilerParams(needs_layout_passes=False),
    )
    def sc(inds_hbm, q_hbm, sx_hbm, sy_hbm, sz_hbm, feats_hbm, mods_hbm,
           w_hbm, kp_hbm, out_hbm,
           sxv, syv, szv, wv, kpv, *slots):
        cc = lax.axis_index("c")
        ss = lax.axis_index("s")
        pltpu.sync_copy(sx_hbm, sxv)
        pltpu.sync_copy(sy_hbm, syv)
        pltpu.sync_copy(sz_hbm, szv)
        pltpu.sync_copy(w_hbm, wv)
        pltpu.sync_copy(kp_hbm, kpv)
        kxv = kpv[pl.ds(0, 16)]
        kyv = kpv[pl.ds(16, 16)]
        kzv = kpv[pl.ds(32, 16)]
        kxs = [kxv[k] for k in range(_K)]
        kys = [kyv[k] for k in range(_K)]
        kzs = [kzv[k] for k in range(_K)]
        chunks = jnp.where(cc == 0, chunks0, chunks1)
        pairs = chunks // 2
        base = jnp.where(
            cc == 0,
            ss * (chunks0 * _R),
            _NS * chunks0 * _R + ss * (chunks1 * _R))
        slot0 = slots[:9]
        slot1 = slots[9:]

        def idx_copies(r0, indv, modsv, qv, isem):
            e0 = r0 * _H
            return (
                pltpu.make_async_copy(inds_hbm.at[pl.ds(e0, 128)],
                                      indv.at[0], isem),
                pltpu.make_async_copy(inds_hbm.at[pl.ds(e0 + 128, 128)],
                                      indv.at[1], isem),
                pltpu.make_async_copy(mods_hbm.at[pl.ds(r0, _R)], modsv, isem),
                pltpu.make_async_copy(q_hbm.at[pl.ds(r0, _R)], qv, isem),
            )

        def gather_copies(indv, featv, gsem):
            return (
                pltpu.make_async_copy(feats_hbm.at[indv.at[0]],
                                      featv.at[pl.ds(0, 128)], gsem),
                pltpu.make_async_copy(feats_hbm.at[indv.at[1]],
                                      featv.at[pl.ds(128, 128)], gsem),
            )

        def compute(ci, slot, nslot):
            (indv, featv, modsv, qv, cbv, outv, isem, gsem, osem) = slot
            (nindv, nfeatv, nmodsv, nqv, _ncbv, _noutv,
             nisem, ngsem, _nosem) = nslot
            r0 = base + ci * _R
            for cp in gather_copies(indv, featv, gsem):
                cp.wait()

            @pl.when(ci + 1 < chunks)
            def _():
                for cp in idx_copies(r0 + _R, nindv, nmodsv, nqv, nisem):
                    cp.wait()
                for cp in gather_copies(nindv, nfeatv, ngsem):
                    cp.start()

            @pl.when(ci >= 2)
            def _():
                pltpu.make_async_copy(outv, out_hbm.at[pl.ds(r0, _R)],
                                      osem).wait()

            def geo(b, c2):
                idx = indv[b >> 3, pl.ds((b & 7) * 16, 16)]
                xs = plsc.load_gather(sxv, [idx])
                ys = plsc.load_gather(syv, [idx])
                zs = plsc.load_gather(szv, [idx])
                rb = b >> 1
                qrow = qv[rb, pl.ds(0, 16)]
                dx = xs - qrow[0]
                dy = ys - qrow[1]
                dz = zs - qrow[2]
                best = jnp.full((16,), 1.0e30, jnp.float32)
                bi = jnp.zeros((16,), jnp.int32)
                for k in range(_K):
                    ddx = dx - kxs[k]
                    ddy = dy - kys[k]
                    ddz = dz - kzs[k]
                    sq = ddx * ddx + ddy * ddy + ddz * ddz
                    u = sq < best
                    best = jnp.where(u, sq, best)
                    bi = jnp.where(u, k, bi)
                a = jnp.maximum(best, 1.0e-20)
                ii = plsc.bitcast(a, jnp.int32)
                y = plsc.bitcast(0x5F3759DF - (ii >> 1), jnp.float32)
                y = y * (1.5 - 0.5 * a * y * y)
                y = y * (1.5 - 0.5 * a * y * y)
                y = y * (1.5 - 0.5 * a * y * y)
                d = a * y
                infl = jnp.maximum(1.0 - d * (1.0 / _SIGMA), 0.0)
                combo = (plsc.bitcast(infl, jnp.int32) & jnp.int32(-16)) | bi
                cbv[pl.ds(b * 16, 16)] = combo
                return c2

            lax.fori_loop(0, _E // 16, geo, 0)

            def row(r, c2):
                def edge(h, accs):
                    e = r * _H + h
                    s = cbv[pl.ds(e, 16)][0]
                    nn = s & 0xF
                    fvec = plsc.bitcast(
                        jnp.full((16,), s, jnp.int32) & jnp.int32(-16),
                        jnp.float32)
                    cvec = modsv[r, pl.ds(nn * _CPG, _CPG)] * fvec
                    return tuple(
                        accs[g]
                        + featv[e, pl.ds(g * _CPG, _CPG)]
                        * (wv[nn, pl.ds(g * _CPG, _CPG)] * cvec)
                        for g in range(_GROUPS))

                accs = lax.fori_loop(
                    0, _H, edge,
                    tuple(jnp.zeros((_CPG,), jnp.float32)
                          for _ in range(_GROUPS)))
                for g in range(_GROUPS):
                    outv[r, pl.ds(g * _CPG, _CPG)] = accs[g]
                return c2

            lax.fori_loop(0, _R, row, 0)
            pltpu.async_copy(outv, out_hbm.at[pl.ds(r0, _R)], osem)

            @pl.when(ci + 2 < chunks)
            def _():
                for cp in idx_copies(base + (ci + 2) * _R,
                                     indv, modsv, qv, isem):
                    cp.start()

        for cp in idx_copies(base, slot0[0], slot0[2], slot0[3], slot0[6]):
            cp.start()
        for cp in idx_copies(base + _R, slot1[0], slot1[2], slot1[3],
                             slot1[6]):
            cp.start()
        for cp in idx_copies(base, slot0[0], slot0[2], slot0[3], slot0[6]):
            cp.wait()
        for cp in gather_copies(slot0[0], slot0[1], slot0[7]):
            cp.start()

        def pair(p, carry):
            compute(2 * p, slot0, slot1)
            compute(2 * p + 1, slot1, slot0)
            return carry

        lax.fori_loop(0, pairs, pair, 0)
        pltpu.make_async_copy(
            slot0[5], out_hbm.at[pl.ds(base + (chunks - 2) * _R, _R)],
            slot0[8]).wait()
        pltpu.make_async_copy(
            slot1[5], out_hbm.at[pl.ds(base + (chunks - 1) * _R, _R)],
            slot1[8]).wait()

    return sc


def kernel(q_pts, s_pts, s_feats, neighb_inds, weights, W1, b1, W2, kernel_points):
    N = s_feats.shape[0]
    NP = 16 * (_CHUNKS0 + _CHUNKS1) * _R
    pad = NP - N

    sf_pad = jnp.pad(s_feats, ((0, pad), (0, 0)))
    W2p = jnp.pad(W2, ((0, 0), (0, 256 - W2.shape[1])))
    mods = _mods_tc(sf_pad, W1, b1.reshape(1, _C), W2p)

    inds = jnp.pad(neighb_inds, ((0, pad), (0, 0))).reshape(NP * _H)
    qp = jnp.pad(q_pts, ((0, pad), (0, 13)))
    sx = s_pts[:, 0]
    sy = s_pts[:, 1]
    sz = s_pts[:, 2]
    kp = jnp.pad(kernel_points.T, ((0, 0), (0, 1))).reshape(48)

    out = _make_sc(N, NP, _CHUNKS0, _CHUNKS1)(
        inds, qp, sx, sy, sz, s_feats, mods, weights, kp)
    return out[:N]


# speedup vs baseline: 1.0123x; 1.0123x over previous
import functools
import jax
import jax.numpy as jnp
from jax import lax
from jax.experimental import pallas as pl
from jax.experimental.pallas import tpu as pltpu
from jax.experimental.pallas import tpu_sc as plsc

_NC = 2
_NS = 16
_NW = _NC * _NS

_H = 32
_C = 128
_K = 15
_CPG = 16
_GROUPS = 8
_SIGMA = 2.0

_R = 8
_E = _R * _H

_CHUNKS0 = 66
_CHUNKS1 = 14


def _mods_tc(sf_pad, W1, b1r, W2p):
    NP = sf_pad.shape[0]
    BR = 256

    def body(x_ref, w1_ref, b1_ref, w2_ref, o_ref):
        h = jnp.dot(x_ref[...], w1_ref[...], preferred_element_type=jnp.float32)
        h = h + b1_ref[...]
        h = jnp.where(h >= 0.0, h, 0.1 * h)
        m = jnp.dot(h, w2_ref[...], preferred_element_type=jnp.float32)
        o_ref[...] = jax.nn.sigmoid(m)

    return pl.pallas_call(
        body,
        grid=(NP // BR,),
        in_specs=[
            pl.BlockSpec((BR, _C), lambda i: (i, 0)),
            pl.BlockSpec((_C, _C), lambda i: (0, 0)),
            pl.BlockSpec((1, _C), lambda i: (0, 0)),
            pl.BlockSpec((_C, 256), lambda i: (0, 0)),
        ],
        out_specs=pl.BlockSpec((BR, 256), lambda i: (i, 0)),
        out_shape=jax.ShapeDtypeStruct((NP, 256), jnp.float32),
    )(sf_pad, W1, b1r, W2p)


def _make_sc(N, NP, chunks0, chunks1):
    assert 16 * (chunks0 + chunks1) * _R == NP
    assert chunks0 % 2 == 0 and chunks1 % 2 == 0
    mesh = plsc.VectorSubcoreMesh(core_axis_name="c", subcore_axis_name="s",
                                  num_cores=_NC, num_subcores=_NS)

    slot_types = [
        pltpu.VMEM((2, 128), jnp.int32),
        pltpu.VMEM((_E, _C), jnp.float32),
        pltpu.VMEM((_R, 256), jnp.float32),
        pltpu.VMEM((_R, 16), jnp.float32),
        pltpu.VMEM((_E + 16,), jnp.int32),
        pltpu.VMEM((_R, _C), jnp.float32),
        pltpu.SemaphoreType.DMA,
        pltpu.SemaphoreType.DMA,
        pltpu.SemaphoreType.DMA,
    ]

    @functools.partial(
        pl.kernel,
        out_type=jax.ShapeDtypeStruct((NP, _C), jnp.float32),
        mesh=mesh,
        scratch_types=[
            pltpu.VMEM((N,), jnp.float32),
            pltpu.VMEM((N,), jnp.float32),
            pltpu.VMEM((N,), jnp.float32),
            pltpu.VMEM((_K, _C), jnp.float32),
            pltpu.VMEM((48,), jnp.float32),
        ] + slot_types + slot_types,
        compiler_params=pltpu.CompilerParams(needs_layout_passes=False),
    )
    def sc(inds_hbm, q_hbm, sx_hbm, sy_hbm, sz_hbm, feats_hbm, mods_hbm,
           w_hbm, kp_hbm, out_hbm,
           sxv, syv, szv, wv, kpv, *slots):
        cc = lax.axis_index("c")
        ss = lax.axis_index("s")
        pltpu.sync_copy(sx_hbm, sxv)
        pltpu.sync_copy(sy_hbm, syv)
        pltpu.sync_copy(sz_hbm, szv)
        pltpu.sync_copy(w_hbm, wv)
        pltpu.sync_copy(kp_hbm, kpv)
        kxv = kpv[pl.ds(0, 16)]
        kyv = kpv[pl.ds(16, 16)]
        kzv = kpv[pl.ds(32, 16)]
        kxs = [kxv[k] for k in range(_K)]
        kys = [kyv[k] for k in range(_K)]
        kzs = [kzv[k] for k in range(_K)]
        chunks = jnp.where(cc == 0, chunks0, chunks1)
        pairs = chunks // 2
        base = jnp.where(
            cc == 0,
            ss * (chunks0 * _R),
            _NS * chunks0 * _R + ss * (chunks1 * _R))
        slot0 = slots[:9]
        slot1 = slots[9:]

        def idx_copies(r0, indv, modsv, qv, isem):
            e0 = r0 * _H
            return (
                pltpu.make_async_copy(inds_hbm.at[pl.ds(e0, 128)],
                                      indv.at[0], isem),
                pltpu.make_async_copy(inds_hbm.at[pl.ds(e0 + 128, 128)],
                                      indv.at[1], isem),
                pltpu.make_async_copy(mods_hbm.at[pl.ds(r0, _R)], modsv, isem),
                pltpu.make_async_copy(q_hbm.at[pl.ds(r0, _R)], qv, isem),
            )

        def gather_copies(indv, featv, gsem):
            return (
                pltpu.make_async_copy(feats_hbm.at[indv.at[0]],
                                      featv.at[pl.ds(0, 128)], gsem),
                pltpu.make_async_copy(feats_hbm.at[indv.at[1]],
                                      featv.at[pl.ds(128, 128)], gsem),
            )

        def compute(ci, slot, nslot):
            (indv, featv, modsv, qv, cbv, outv, isem, gsem, osem) = slot
            (nindv, nfeatv, nmodsv, nqv, _ncbv, _noutv,
             nisem, ngsem, _nosem) = nslot
            r0 = base + ci * _R
            for cp in gather_copies(indv, featv, gsem):
                cp.wait()

            @pl.when(ci + 1 < chunks)
            def _():
                for cp in idx_copies(r0 + _R, nindv, nmodsv, nqv, nisem):
                    cp.wait()
                for cp in gather_copies(nindv, nfeatv, ngsem):
                    cp.start()

            @pl.when(ci >= 2)
            def _():
                pltpu.make_async_copy(outv, out_hbm.at[pl.ds(r0, _R)],
                                      osem).wait()

            def geo(b, c2):
                idx = indv[b >> 3, pl.ds((b & 7) * 16, 16)]
                xs = plsc.load_gather(sxv, [idx])
                ys = plsc.load_gather(syv, [idx])
                zs = plsc.load_gather(szv, [idx])
                rb = b >> 1
                qrow = qv[rb, pl.ds(0, 16)]
                dx = xs - qrow[0]
                dy = ys - qrow[1]
                dz = zs - qrow[2]
                best = jnp.full((16,), 1.0e30, jnp.float32)
                bi = jnp.zeros((16,), jnp.int32)
                for k in range(_K):
                    ddx = dx - kxs[k]
                    ddy = dy - kys[k]
                    ddz = dz - kzs[k]
                    sq = ddx * ddx + ddy * ddy + ddz * ddz
                    u = sq < best
                    best = jnp.where(u, sq, best)
                    bi = jnp.where(u, k, bi)
                a = jnp.maximum(best, 1.0e-20)
                ii = plsc.bitcast(a, jnp.int32)
                y = plsc.bitcast(0x5F3759DF - (ii >> 1), jnp.float32)
                y = y * (1.5 - 0.5 * a * y * y)
                y = y * (1.5 - 0.5 * a * y * y)
                y = y * (1.5 - 0.5 * a * y * y)
                d = a * y
                infl = jnp.maximum(1.0 - d * (1.0 / _SIGMA), 0.0)
                combo = (plsc.bitcast(infl, jnp.int32) & jnp.int32(-16)) | bi
                cbv[pl.ds(b * 16, 16)] = combo
                return c2

            lax.fori_loop(0, _E // 16, geo, 0)

            def row(r, c2):
                def edge(h, accs):
                    e = r * _H + h
                    s = cbv[pl.ds(e, 16)][0]
                    nn = s & 0xF
                    fvec = plsc.bitcast(
                        jnp.full((16,), s, jnp.int32) & jnp.int32(-16),
                        jnp.float32)
                    cvec = modsv[r, pl.ds(nn * _CPG, _CPG)] * fvec
                    return tuple(
                        accs[g]
                        + featv[e, pl.ds(g * _CPG, _CPG)]
                        * (wv[nn, pl.ds(g * _CPG, _CPG)] * cvec)
                        for g in range(_GROUPS))

                accs = lax.fori_loop(
                    0, _H, edge,
                    tuple(jnp.zeros((_CPG,), jnp.float32)
                          for _ in range(_GROUPS)))
                for g in range(_GROUPS):
                    outv[r, pl.ds(g * _CPG, _CPG)] = accs[g]
                return c2

            lax.fori_loop(0, _R, row, 0)
            pltpu.async_copy(outv, out_hbm.at[pl.ds(r0, _R)], osem)

            @pl.when(ci + 2 < chunks)
            def _():
                for cp in idx_copies(base + (ci + 2) * _R,
                                     indv, modsv, qv, isem):
                    cp.start()

        for cp in idx_copies(base, slot0[0], slot0[2], slot0[3], slot0[6]):
            cp.start()
        for cp in idx_copies(base + _R, slot1[0], slot1[2], slot1[3],
                             slot1[6]):
            cp.start()
        for cp in idx_copies(base, slot0[0], slot0[2], slot0[3], slot0[6]):
            cp.wait()
        for cp in gather_copies(slot0[0], slot0[1], slot0[7]):
            cp.start()

        def pair(p, carry):
            compute(2 * p, slot0, slot1)
            compute(2 * p + 1, slot1, slot0)
            return carry

        lax.fori_loop(0, pairs, pair, 0)
        pltpu.make_async_copy(
            slot0[5], out_hbm.at[pl.ds(base + (chunks - 2) * _R, _R)],
            slot0[8]).wait()
        pltpu.make_async_copy(
            slot1[5], out_hbm.at[pl.ds(base + (chunks - 1) * _R, _R)],
            slot1[8]).wait()

    return sc


def kernel(q_pts, s_pts, s_feats, neighb_inds, weights, W1, b1, W2, kernel_points):
    N = s_feats.shape[0]
    NP = 16 * (_CHUNKS0 + _CHUNKS1) * _R
    pad = NP - N

    sf_pad = jnp.pad(s_feats, ((0, pad), (0, 0)))
    W2p = jnp.pad(W2, ((0, 0), (0, 256 - W2.shape[1])))
    mods = _mods_tc(sf_pad, W1, b1.reshape(1, _C), W2p)

    inds = jnp.pad(neighb_inds, ((0, pad), (0, 0))).reshape(NP * _H)
    qp = jnp.pad(q_pts, ((0, pad), (0, 13)))
    sx = s_pts[:, 0]
    sy = s_pts[:, 1]
    sz = s_pts[:, 2]
    kp = jnp.pad(kernel_points.T, ((0, 0), (0, 1))).reshape(48)

    out = _make_sc(N, NP, _CHUNKS0, _CHUNKS1)(
        inds, qp, sx, sy, sz, s_feats, mods, weights, kp)
    return out[:N]


# speedup vs baseline: 1.0584x; 1.0455x over previous
import functools
import jax
import jax.numpy as jnp
from jax import lax
from jax.experimental import pallas as pl
from jax.experimental.pallas import tpu as pltpu
from jax.experimental.pallas import tpu_sc as plsc

_NC = 2
_NS = 16
_NW = _NC * _NS

_H = 32
_C = 128
_K = 15
_CPG = 16
_GROUPS = 8
_SIGMA = 2.0

_R = 8
_E = _R * _H

_CHUNKS0 = 70
_CHUNKS1 = 10


def _mods_tc(sf_pad, W1, b1r, W2p):
    NP = sf_pad.shape[0]
    BR = 256

    def body(x_ref, w1_ref, b1_ref, w2_ref, o_ref):
        h = jnp.dot(x_ref[...], w1_ref[...], preferred_element_type=jnp.float32)
        h = h + b1_ref[...]
        h = jnp.where(h >= 0.0, h, 0.1 * h)
        m = jnp.dot(h, w2_ref[...], preferred_element_type=jnp.float32)
        o_ref[...] = jax.nn.sigmoid(m)

    return pl.pallas_call(
        body,
        grid=(NP // BR,),
        in_specs=[
            pl.BlockSpec((BR, _C), lambda i: (i, 0)),
            pl.BlockSpec((_C, _C), lambda i: (0, 0)),
            pl.BlockSpec((1, _C), lambda i: (0, 0)),
            pl.BlockSpec((_C, 256), lambda i: (0, 0)),
        ],
        out_specs=pl.BlockSpec((BR, 256), lambda i: (i, 0)),
        out_shape=jax.ShapeDtypeStruct((NP, 256), jnp.float32),
    )(sf_pad, W1, b1r, W2p)


def _make_sc(N, NP, chunks0, chunks1):
    assert 16 * (chunks0 + chunks1) * _R == NP
    assert chunks0 % 2 == 0 and chunks1 % 2 == 0
    mesh = plsc.VectorSubcoreMesh(core_axis_name="c", subcore_axis_name="s",
                                  num_cores=_NC, num_subcores=_NS)

    slot_types = [
        pltpu.VMEM((2, 128), jnp.int32),
        pltpu.VMEM((_E, _C), jnp.float32),
        pltpu.VMEM((_R, 256), jnp.float32),
        pltpu.VMEM((_R, 16), jnp.float32),
        pltpu.VMEM((_E + 16,), jnp.int32),
        pltpu.VMEM((_R, _C), jnp.float32),
        pltpu.SemaphoreType.DMA,
        pltpu.SemaphoreType.DMA,
        pltpu.SemaphoreType.DMA,
    ]

    @functools.partial(
        pl.kernel,
        out_type=jax.ShapeDtypeStruct((NP, _C), jnp.float32),
        mesh=mesh,
        scratch_types=[
            pltpu.VMEM((N,), jnp.float32),
            pltpu.VMEM((N,), jnp.float32),
            pltpu.VMEM((N,), jnp.float32),
            pltpu.VMEM((_K, _C), jnp.float32),
            pltpu.VMEM((48,), jnp.float32),
        ] + slot_types + slot_types,
        compiler_params=pltpu.CompilerParams(needs_layout_passes=False),
    )
    def sc(inds_hbm, q_hbm, sx_hbm, sy_hbm, sz_hbm, feats_hbm, mods_hbm,
           w_hbm, kp_hbm, out_hbm,
           sxv, syv, szv, wv, kpv, *slots):
        cc = lax.axis_index("c")
        ss = lax.axis_index("s")
        pltpu.sync_copy(sx_hbm, sxv)
        pltpu.sync_copy(sy_hbm, syv)
        pltpu.sync_copy(sz_hbm, szv)
        pltpu.sync_copy(w_hbm, wv)
        pltpu.sync_copy(kp_hbm, kpv)
        kxv = kpv[pl.ds(0, 16)]
        kyv = kpv[pl.ds(16, 16)]
        kzv = kpv[pl.ds(32, 16)]
        kxs = [kxv[k] for k in range(_K)]
        kys = [kyv[k] for k in range(_K)]
        kzs = [kzv[k] for k in range(_K)]
        chunks = jnp.where(cc == 0, chunks0, chunks1)
        pairs = chunks // 2
        base = jnp.where(
            cc == 0,
            ss * (chunks0 * _R),
            _NS * chunks0 * _R + ss * (chunks1 * _R))
        slot0 = slots[:9]
        slot1 = slots[9:]

        def idx_copies(r0, indv, modsv, qv, isem):
            e0 = r0 * _H
            return (
                pltpu.make_async_copy(inds_hbm.at[pl.ds(e0, 128)],
                                      indv.at[0], isem),
                pltpu.make_async_copy(inds_hbm.at[pl.ds(e0 + 128, 128)],
                                      indv.at[1], isem),
                pltpu.make_async_copy(mods_hbm.at[pl.ds(r0, _R)], modsv, isem),
                pltpu.make_async_copy(q_hbm.at[pl.ds(r0, _R)], qv, isem),
            )

        def gather_copies(indv, featv, gsem):
            return (
                pltpu.make_async_copy(feats_hbm.at[indv.at[0]],
                                      featv.at[pl.ds(0, 128)], gsem),
                pltpu.make_async_copy(feats_hbm.at[indv.at[1]],
                                      featv.at[pl.ds(128, 128)], gsem),
            )

        def compute(ci, slot, nslot):
            (indv, featv, modsv, qv, cbv, outv, isem, gsem, osem) = slot
            (nindv, nfeatv, nmodsv, nqv, _ncbv, _noutv,
             nisem, ngsem, _nosem) = nslot
            r0 = base + ci * _R
            for cp in gather_copies(indv, featv, gsem):
                cp.wait()

            @pl.when(ci + 1 < chunks)
            def _():
                for cp in idx_copies(r0 + _R, nindv, nmodsv, nqv, nisem):
                    cp.wait()
                for cp in gather_copies(nindv, nfeatv, ngsem):
                    cp.start()

            @pl.when(ci >= 2)
            def _():
                pltpu.make_async_copy(outv, out_hbm.at[pl.ds(r0, _R)],
                                      osem).wait()

            def geo(b, c2):
                idx = indv[b >> 3, pl.ds((b & 7) * 16, 16)]
                xs = plsc.load_gather(sxv, [idx])
                ys = plsc.load_gather(syv, [idx])
                zs = plsc.load_gather(szv, [idx])
                rb = b >> 1
                qrow = qv[rb, pl.ds(0, 16)]
                dx = xs - qrow[0]
                dy = ys - qrow[1]
                dz = zs - qrow[2]
                best = jnp.full((16,), 1.0e30, jnp.float32)
                bi = jnp.zeros((16,), jnp.int32)
                for k in range(_K):
                    ddx = dx - kxs[k]
                    ddy = dy - kys[k]
                    ddz = dz - kzs[k]
                    sq = ddx * ddx + ddy * ddy + ddz * ddz
                    u = sq < best
                    best = jnp.where(u, sq, best)
                    bi = jnp.where(u, k, bi)
                a = jnp.maximum(best, 1.0e-20)
                ii = plsc.bitcast(a, jnp.int32)
                y = plsc.bitcast(0x5F3759DF - (ii >> 1), jnp.float32)
                y = y * (1.5 - 0.5 * a * y * y)
                y = y * (1.5 - 0.5 * a * y * y)
                y = y * (1.5 - 0.5 * a * y * y)
                d = a * y
                infl = jnp.maximum(1.0 - d * (1.0 / _SIGMA), 0.0)
                combo = (plsc.bitcast(infl, jnp.int32) & jnp.int32(-16)) | bi
                cbv[pl.ds(b * 16, 16)] = combo
                return c2

            lax.fori_loop(0, _E // 16, geo, 0)

            def row(r, c2):
                def edge(h, accs):
                    e = r * _H + h
                    s = cbv[pl.ds(e, 16)][0]
                    nn = s & 0xF
                    fvec = plsc.bitcast(
                        jnp.full((16,), s, jnp.int32) & jnp.int32(-16),
                        jnp.float32)
                    cvec = modsv[r, pl.ds(nn * _CPG, _CPG)] * fvec
                    return tuple(
                        accs[g]
                        + featv[e, pl.ds(g * _CPG, _CPG)]
                        * (wv[nn, pl.ds(g * _CPG, _CPG)] * cvec)
                        for g in range(_GROUPS))

                accs = lax.fori_loop(
                    0, _H, edge,
                    tuple(jnp.zeros((_CPG,), jnp.float32)
                          for _ in range(_GROUPS)))
                for g in range(_GROUPS):
                    outv[r, pl.ds(g * _CPG, _CPG)] = accs[g]
                return c2

            lax.fori_loop(0, _R, row, 0)
            pltpu.async_copy(outv, out_hbm.at[pl.ds(r0, _R)], osem)

            @pl.when(ci + 2 < chunks)
            def _():
                for cp in idx_copies(base + (ci + 2) * _R,
                                     indv, modsv, qv, isem):
                    cp.start()

        for cp in idx_copies(base, slot0[0], slot0[2], slot0[3], slot0[6]):
            cp.start()
        for cp in idx_copies(base + _R, slot1[0], slot1[2], slot1[3],
                             slot1[6]):
            cp.start()
        for cp in idx_copies(base, slot0[0], slot0[2], slot0[3], slot0[6]):
            cp.wait()
        for cp in gather_copies(slot0[0], slot0[1], slot0[7]):
            cp.start()

        def pair(p, carry):
            compute(2 * p, slot0, slot1)
            compute(2 * p + 1, slot1, slot0)
            return carry

        lax.fori_loop(0, pairs, pair, 0)
        pltpu.make_async_copy(
            slot0[5], out_hbm.at[pl.ds(base + (chunks - 2) * _R, _R)],
            slot0[8]).wait()
        pltpu.make_async_copy(
            slot1[5], out_hbm.at[pl.ds(base + (chunks - 1) * _R, _R)],
            slot1[8]).wait()

    return sc


def kernel(q_pts, s_pts, s_feats, neighb_inds, weights, W1, b1, W2, kernel_points):
    N = s_feats.shape[0]
    NP = 16 * (_CHUNKS0 + _CHUNKS1) * _R
    pad = NP - N

    sf_pad = jnp.pad(s_feats, ((0, pad), (0, 0)))
    W2p = jnp.pad(W2, ((0, 0), (0, 256 - W2.shape[1])))
    mods = _mods_tc(sf_pad, W1, b1.reshape(1, _C), W2p)

    inds = jnp.pad(neighb_inds, ((0, pad), (0, 0))).reshape(NP * _H)
    qp = jnp.pad(q_pts, ((0, pad), (0, 13)))
    sx = s_pts[:, 0]
    sy = s_pts[:, 1]
    sz = s_pts[:, 2]
    kp = jnp.pad(kernel_points.T, ((0, 0), (0, 1))).reshape(48)

    out = _make_sc(N, NP, _CHUNKS0, _CHUNKS1)(
        inds, qp, sx, sy, sz, s_feats, mods, weights, kp)
    return out[:N]


# speedup vs baseline: 1.0996x; 1.0389x over previous
import functools
import jax
import jax.numpy as jnp
from jax import lax
from jax.experimental import pallas as pl
from jax.experimental.pallas import tpu as pltpu
from jax.experimental.pallas import tpu_sc as plsc

_NC = 2
_NS = 16
_NW = _NC * _NS

_H = 32
_C = 128
_K = 15
_CPG = 16
_GROUPS = 8
_SIGMA = 2.0

_R = 8
_E = _R * _H

_CHUNKS0 = 74
_CHUNKS1 = 6


def _mods_tc(sf_pad, W1, b1r, W2p):
    NP = sf_pad.shape[0]
    BR = 256

    def body(x_ref, w1_ref, b1_ref, w2_ref, o_ref):
        h = jnp.dot(x_ref[...], w1_ref[...], preferred_element_type=jnp.float32)
        h = h + b1_ref[...]
        h = jnp.where(h >= 0.0, h, 0.1 * h)
        m = jnp.dot(h, w2_ref[...], preferred_element_type=jnp.float32)
        o_ref[...] = jax.nn.sigmoid(m)

    return pl.pallas_call(
        body,
        grid=(NP // BR,),
        in_specs=[
            pl.BlockSpec((BR, _C), lambda i: (i, 0)),
            pl.BlockSpec((_C, _C), lambda i: (0, 0)),
            pl.BlockSpec((1, _C), lambda i: (0, 0)),
            pl.BlockSpec((_C, 256), lambda i: (0, 0)),
        ],
        out_specs=pl.BlockSpec((BR, 256), lambda i: (i, 0)),
        out_shape=jax.ShapeDtypeStruct((NP, 256), jnp.float32),
    )(sf_pad, W1, b1r, W2p)


def _make_sc(N, NP, chunks0, chunks1):
    assert 16 * (chunks0 + chunks1) * _R == NP
    assert chunks0 % 2 == 0 and chunks1 % 2 == 0
    mesh = plsc.VectorSubcoreMesh(core_axis_name="c", subcore_axis_name="s",
                                  num_cores=_NC, num_subcores=_NS)

    slot_types = [
        pltpu.VMEM((2, 128), jnp.int32),
        pltpu.VMEM((_E, _C), jnp.float32),
        pltpu.VMEM((_R, 256), jnp.float32),
        pltpu.VMEM((_R, 16), jnp.float32),
        pltpu.VMEM((_E + 16,), jnp.int32),
        pltpu.VMEM((_R, _C), jnp.float32),
        pltpu.SemaphoreType.DMA,
        pltpu.SemaphoreType.DMA,
        pltpu.SemaphoreType.DMA,
    ]

    @functools.partial(
        pl.kernel,
        out_type=jax.ShapeDtypeStruct((NP, _C), jnp.float32),
        mesh=mesh,
        scratch_types=[
            pltpu.VMEM((N,), jnp.float32),
            pltpu.VMEM((N,), jnp.float32),
            pltpu.VMEM((N,), jnp.float32),
            pltpu.VMEM((_K, _C), jnp.float32),
            pltpu.VMEM((48,), jnp.float32),
        ] + slot_types + slot_types,
        compiler_params=pltpu.CompilerParams(needs_layout_passes=False),
    )
    def sc(inds_hbm, q_hbm, sx_hbm, sy_hbm, sz_hbm, feats_hbm, mods_hbm,
           w_hbm, kp_hbm, out_hbm,
           sxv, syv, szv, wv, kpv, *slots):
        cc = lax.axis_index("c")
        ss = lax.axis_index("s")
        pltpu.sync_copy(sx_hbm, sxv)
        pltpu.sync_copy(sy_hbm, syv)
        pltpu.sync_copy(sz_hbm, szv)
        pltpu.sync_copy(w_hbm, wv)
        pltpu.sync_copy(kp_hbm, kpv)
        kxv = kpv[pl.ds(0, 16)]
        kyv = kpv[pl.ds(16, 16)]
        kzv = kpv[pl.ds(32, 16)]
        kxs = [kxv[k] for k in range(_K)]
        kys = [kyv[k] for k in range(_K)]
        kzs = [kzv[k] for k in range(_K)]
        chunks = jnp.where(cc == 0, chunks0, chunks1)
        pairs = chunks // 2
        base = jnp.where(
            cc == 0,
            ss * (chunks0 * _R),
            _NS * chunks0 * _R + ss * (chunks1 * _R))
        slot0 = slots[:9]
        slot1 = slots[9:]

        def idx_copies(r0, indv, modsv, qv, isem):
            e0 = r0 * _H
            return (
                pltpu.make_async_copy(inds_hbm.at[pl.ds(e0, 128)],
                                      indv.at[0], isem),
                pltpu.make_async_copy(inds_hbm.at[pl.ds(e0 + 128, 128)],
                                      indv.at[1], isem),
                pltpu.make_async_copy(mods_hbm.at[pl.ds(r0, _R)], modsv, isem),
                pltpu.make_async_copy(q_hbm.at[pl.ds(r0, _R)], qv, isem),
            )

        def gather_copies(indv, featv, gsem):
            return (
                pltpu.make_async_copy(feats_hbm.at[indv.at[0]],
                                      featv.at[pl.ds(0, 128)], gsem),
                pltpu.make_async_copy(feats_hbm.at[indv.at[1]],
                                      featv.at[pl.ds(128, 128)], gsem),
            )

        def compute(ci, slot, nslot):
            (indv, featv, modsv, qv, cbv, outv, isem, gsem, osem) = slot
            (nindv, nfeatv, nmodsv, nqv, _ncbv, _noutv,
             nisem, ngsem, _nosem) = nslot
            r0 = base + ci * _R
            for cp in gather_copies(indv, featv, gsem):
                cp.wait()

            @pl.when(ci + 1 < chunks)
            def _():
                for cp in idx_copies(r0 + _R, nindv, nmodsv, nqv, nisem):
                    cp.wait()
                for cp in gather_copies(nindv, nfeatv, ngsem):
                    cp.start()

            @pl.when(ci >= 2)
            def _():
                pltpu.make_async_copy(outv, out_hbm.at[pl.ds(r0, _R)],
                                      osem).wait()

            def geo(b, c2):
                idx = indv[b >> 3, pl.ds((b & 7) * 16, 16)]
                xs = plsc.load_gather(sxv, [idx])
                ys = plsc.load_gather(syv, [idx])
                zs = plsc.load_gather(szv, [idx])
                rb = b >> 1
                qrow = qv[rb, pl.ds(0, 16)]
                dx = xs - qrow[0]
                dy = ys - qrow[1]
                dz = zs - qrow[2]
                best = jnp.full((16,), 1.0e30, jnp.float32)
                bi = jnp.zeros((16,), jnp.int32)
                for k in range(_K):
                    ddx = dx - kxs[k]
                    ddy = dy - kys[k]
                    ddz = dz - kzs[k]
                    sq = ddx * ddx + ddy * ddy + ddz * ddz
                    u = sq < best
                    best = jnp.where(u, sq, best)
                    bi = jnp.where(u, k, bi)
                a = jnp.maximum(best, 1.0e-20)
                ii = plsc.bitcast(a, jnp.int32)
                y = plsc.bitcast(0x5F3759DF - (ii >> 1), jnp.float32)
                y = y * (1.5 - 0.5 * a * y * y)
                y = y * (1.5 - 0.5 * a * y * y)
                y = y * (1.5 - 0.5 * a * y * y)
                d = a * y
                infl = jnp.maximum(1.0 - d * (1.0 / _SIGMA), 0.0)
                combo = (plsc.bitcast(infl, jnp.int32) & jnp.int32(-16)) | bi
                cbv[pl.ds(b * 16, 16)] = combo
                return c2

            lax.fori_loop(0, _E // 16, geo, 0)

            def row(r, c2):
                def edge(h, accs):
                    e = r * _H + h
                    s = cbv[pl.ds(e, 16)][0]
                    nn = s & 0xF
                    fvec = plsc.bitcast(
                        jnp.full((16,), s, jnp.int32) & jnp.int32(-16),
                        jnp.float32)
                    cvec = modsv[r, pl.ds(nn * _CPG, _CPG)] * fvec
                    return tuple(
                        accs[g]
                        + featv[e, pl.ds(g * _CPG, _CPG)]
                        * (wv[nn, pl.ds(g * _CPG, _CPG)] * cvec)
                        for g in range(_GROUPS))

                accs = lax.fori_loop(
                    0, _H, edge,
                    tuple(jnp.zeros((_CPG,), jnp.float32)
                          for _ in range(_GROUPS)))
                for g in range(_GROUPS):
                    outv[r, pl.ds(g * _CPG, _CPG)] = accs[g]
                return c2

            lax.fori_loop(0, _R, row, 0)
            pltpu.async_copy(outv, out_hbm.at[pl.ds(r0, _R)], osem)

            @pl.when(ci + 2 < chunks)
            def _():
                for cp in idx_copies(base + (ci + 2) * _R,
                                     indv, modsv, qv, isem):
                    cp.start()

        for cp in idx_copies(base, slot0[0], slot0[2], slot0[3], slot0[6]):
            cp.start()
        for cp in idx_copies(base + _R, slot1[0], slot1[2], slot1[3],
                             slot1[6]):
            cp.start()
        for cp in idx_copies(base, slot0[0], slot0[2], slot0[3], slot0[6]):
            cp.wait()
        for cp in gather_copies(slot0[0], slot0[1], slot0[7]):
            cp.start()

        def pair(p, carry):
            compute(2 * p, slot0, slot1)
            compute(2 * p + 1, slot1, slot0)
            return carry

        lax.fori_loop(0, pairs, pair, 0)
        pltpu.make_async_copy(
            slot0[5], out_hbm.at[pl.ds(base + (chunks - 2) * _R, _R)],
            slot0[8]).wait()
        pltpu.make_async_copy(
            slot1[5], out_hbm.at[pl.ds(base + (chunks - 1) * _R, _R)],
            slot1[8]).wait()

    return sc


def kernel(q_pts, s_pts, s_feats, neighb_inds, weights, W1, b1, W2, kernel_points):
    N = s_feats.shape[0]
    NP = 16 * (_CHUNKS0 + _CHUNKS1) * _R
    pad = NP - N

    sf_pad = jnp.pad(s_feats, ((0, pad), (0, 0)))
    W2p = jnp.pad(W2, ((0, 0), (0, 256 - W2.shape[1])))
    mods = _mods_tc(sf_pad, W1, b1.reshape(1, _C), W2p)

    inds = jnp.pad(neighb_inds, ((0, pad), (0, 0))).reshape(NP * _H)
    qp = jnp.pad(q_pts, ((0, pad), (0, 13)))
    sx = s_pts[:, 0]
    sy = s_pts[:, 1]
    sz = s_pts[:, 2]
    kp = jnp.pad(kernel_points.T, ((0, 0), (0, 1))).reshape(48)

    out = _make_sc(N, NP, _CHUNKS0, _CHUNKS1)(
        inds, qp, sx, sy, sz, s_feats, mods, weights, kp)
    return out[:N]
